# Initial kernel scaffold; baseline (speedup 1.0000x reference)
import jax
import jax.numpy as jnp
from jax.experimental import pallas as pl


def kernel(x, table, W):
    raise NotImplementedError("write your pallas kernel here")


# speedup vs baseline: 3.0769x; 3.0769x over previous
import functools

import jax
import jax.numpy as jnp
from jax import lax
from jax.experimental import pallas as pl
from jax.experimental.pallas import tpu as pltpu
from jax.experimental.pallas import tpu_sc as plsc

HASH_SIZE = 100000
HASH_DIM = 512
MODEL_DIM = 2048
B = 4
S = 4096
N = B * S

NC, NS = 2, 16
NW = NC * NS
TOK_PER_W = N // NW
GCHUNK = 128
NCHUNK = TOK_PER_W // GCHUNK
LANES = 16

MUL_LO = 2654435761 % HASH_SIZE
MUL_HI = (1024 * 2654435761) % HASH_SIZE


def _sc_hash_gather(x, p1, p2, table):
    mesh = plsc.VectorSubcoreMesh(core_axis_name="c", subcore_axis_name="s")

    @functools.partial(
        pl.kernel,
        mesh=mesh,
        out_type=jax.ShapeDtypeStruct((N, HASH_DIM), jnp.float32),
        scratch_types=[
            pltpu.VMEM((TOK_PER_W,), jnp.int32),
            pltpu.VMEM((TOK_PER_W,), jnp.int32),
            pltpu.VMEM((TOK_PER_W,), jnp.int32),
            pltpu.VMEM((NCHUNK, GCHUNK), jnp.int32),
            pltpu.VMEM((GCHUNK, HASH_DIM), jnp.float32),
            pltpu.SemaphoreType.DMA,
        ],
    )
    def k(x_hbm, p1_hbm, p2_hbm, table_hbm, emb_hbm, x_v, p1_v, p2_v, idx_v, rows_v, sem):
        wid = lax.axis_index("s") * NC + lax.axis_index("c")
        base = wid * TOK_PER_W
        pltpu.sync_copy(x_hbm.at[pl.ds(base, TOK_PER_W)], x_v)
        pltpu.sync_copy(p1_hbm.at[pl.ds(base, TOK_PER_W)], p1_v)
        pltpu.sync_copy(p2_hbm.at[pl.ds(base, TOK_PER_W)], p2_v)

        for c in range(NCHUNK):
            for j in range(GCHUNK // LANES):
                t = c * GCHUNK + j * LANES
                xv = x_v[pl.ds(t, LANES)]
                pv = p1_v[pl.ds(t, LANES)]
                qv = p2_v[pl.ds(t, LANES)]
                v = xv ^ (pv << 1) ^ (qv << 2)
                vhi = v >> 10
                vlo = v & 1023
                h = (vhi * MUL_HI + vlo * MUL_LO) % HASH_SIZE
                idx_v[c, pl.ds(j * LANES, LANES)] = h

        for c in range(NCHUNK):
            pltpu.async_copy(table_hbm.at[idx_v.at[jnp.int32(c)]], rows_v, sem).wait()
            pltpu.sync_copy(rows_v, emb_hbm.at[pl.ds(base + c * GCHUNK, GCHUNK)])

    return k(x, p1, p2, table)


def _mm_body(a_ref, w_ref, o_ref):
    o_ref[...] = lax.dot_general(
        a_ref[...], w_ref[...],
        (((1,), (1,)), ((), ())),
        preferred_element_type=jnp.float32,
    )


def _tc_matmul(emb, W):
    BM = 512
    grid = (N // BM,)
    return pl.pallas_call(
        _mm_body,
        grid=grid,
        in_specs=[
            pl.BlockSpec((BM, HASH_DIM), lambda i: (i, jnp.int32(0))),
            pl.BlockSpec((MODEL_DIM, HASH_DIM), lambda i: (jnp.int32(0), jnp.int32(0))),
        ],
        out_specs=pl.BlockSpec((BM, MODEL_DIM), lambda i: (i, jnp.int32(0))),
        out_shape=jax.ShapeDtypeStruct((N, MODEL_DIM), jnp.float32),
    )(emb, W)


def kernel(x, table, W):
    x32 = x.astype(jnp.int32)
    p1 = jnp.concatenate([x32[:, :1], x32[:, :-1]], axis=1)
    p2 = jnp.concatenate([x32[:, :2], x32[:, :-2]], axis=1)
    emb = _sc_hash_gather(x32.reshape(-1), p1.reshape(-1), p2.reshape(-1), table)
    out = _tc_matmul(emb, W)
    return out.reshape(B, S, MODEL_DIM)
